# baseline (device time: 17373 ns/iter reference)
import os

import jax
import jax.numpy as jnp
from jax import lax
from jax.experimental import pallas as pl
from jax.experimental.pallas import tpu as pltpu

N_DEV = 4
N_PIECES = 4

_PROBE_NO_COMM = os.environ.get("SCB_PROBE_NO_COMM") == "1"


def kernel(x, w_mat):
    m_global, k_per = x.shape
    _, n = w_mat.shape
    m_per = m_global // N_DEV
    n_streams = 2 * N_PIECES
    n_piece = n // n_streams

    def body(x_ref, w_ref, out_ref, sbufs, recvs, ssems, rsems):
        my = lax.axis_index("i")
        left = lax.rem(my + N_DEV - 1, N_DEV)
        right = lax.rem(my + 1, N_DEV)

        if not _PROBE_NO_COMM:
            barrier_sem = pltpu.get_barrier_semaphore()
            for nbr in (left, right):
                pl.semaphore_signal(
                    barrier_sem, inc=1,
                    device_id=(nbr,), device_id_type=pl.DeviceIdType.MESH,
                )

        w = w_ref[:, :].astype(jnp.bfloat16)

        def c_right(s):
            return lax.rem(my - 1 - s + 2 * N_DEV, N_DEV)

        def c_left(s):
            return lax.rem(my + 1 + s, N_DEV)

        streams = []
        for p in range(N_PIECES):
            streams.append(("R", p * n_piece))
            streams.append(("L", n // 2 + p * n_piece))

        def partial_bf16(c, col0):
            xs = x_ref[pl.ds(c * m_per, m_per), :].astype(jnp.bfloat16)
            return lax.dot_general(
                xs, w[:, col0:col0 + n_piece], (((1,), (0,)), ((), ())),
                preferred_element_type=jnp.float32,
            ).astype(jnp.bfloat16)

        def next_partials(s):
            return [
                partial_bf16(c_right(s) if d == "R" else c_left(s), col0)
                for d, col0 in streams
            ]

        rdmas = [[None] * (N_DEV - 1) for _ in streams]

        def start_hop(k, d, s):
            if _PROBE_NO_COMM:
                return
            rdma = pltpu.make_async_remote_copy(
                src_ref=sbufs.at[k], dst_ref=recvs.at[k, s],
                send_sem=ssems.at[k, s], recv_sem=rsems.at[k, s],
                device_id=(right if d == "R" else left,),
                device_id_type=pl.DeviceIdType.MESH,
            )
            rdma.start()
            rdmas[k][s] = rdma

        pbf = [None] * len(streams)
        for k, (d, col0) in enumerate(streams):
            pbf[k] = partial_bf16(c_right(0) if d == "R" else c_left(0), col0)
            sbufs[k, :, :] = pbf[k]
            if k == 0 and not _PROBE_NO_COMM:
                pl.semaphore_wait(barrier_sem, 2)
            start_hop(k, d, 0)
        pbf = next_partials(1)

        for s in range(1, N_DEV - 1):
            for k, (d, _) in enumerate(streams):
                if not _PROBE_NO_COMM:
                    rdmas[k][s - 1].wait_recv()
                val = pbf[k] + recvs[k, s - 1, :, :]
                if not _PROBE_NO_COMM:
                    rdmas[k][s - 1].wait_send()
                sbufs[k, :, :] = val
                start_hop(k, d, s)
            pbf = next_partials(s + 1)

        for k, (_, col0) in enumerate(streams):
            if not _PROBE_NO_COMM:
                rdmas[k][N_DEV - 2].wait_recv()
            out_ref[:, col0:col0 + n_piece] = (
                pbf[k] + recvs[k, N_DEV - 2, :, :]
            ).astype(jnp.float32)
            if not _PROBE_NO_COMM:
                rdmas[k][N_DEV - 2].wait_send()

    return pl.pallas_call(
        body,
        out_shape=jax.ShapeDtypeStruct((m_per, n), jnp.float32),
        in_specs=[
            pl.BlockSpec(memory_space=pltpu.VMEM),
            pl.BlockSpec(memory_space=pltpu.VMEM),
        ],
        out_specs=pl.BlockSpec(memory_space=pltpu.VMEM),
        scratch_shapes=[
            pltpu.VMEM((n_streams, m_per, n_piece), jnp.bfloat16),
            pltpu.VMEM((n_streams, N_DEV - 1, m_per, n_piece), jnp.bfloat16),
            pltpu.SemaphoreType.DMA((n_streams, N_DEV - 1)),
            pltpu.SemaphoreType.DMA((n_streams, N_DEV - 1)),
        ],
        compiler_params=(
            None if _PROBE_NO_COMM else pltpu.CompilerParams(collective_id=0)
        ),
    )(x, w_mat)
